# baseline (speedup 1.0000x reference)
import functools

import jax
import jax.numpy as jnp
from jax import lax
from jax.experimental import pallas as pl
from jax.experimental.pallas import tpu as pltpu
from jax.experimental.pallas import tpu_sc as plsc

_NC = 2
_NS = 16
_NW = _NC * _NS
_L = 16
_K = 96


@functools.cache
def _build_warp(B, H, W, C):
    P = B * H * W
    CP = C // 2
    per_w = P // _NW
    n_chunks = per_w // _K
    assert per_w % _K == 0 and CP % _L == 0 and W % _K == 0
    assert n_chunks % 2 == 0

    mesh = plsc.VectorSubcoreMesh(core_axis_name="c", subcore_axis_name="s")

    def slot_scratch():
        return [
            pltpu.VMEM((2 * _K,), jnp.float32),
            pltpu.VMEM((_K,), jnp.int32),
            pltpu.VMEM((_K,), jnp.int32),
            pltpu.VMEM((_K,), jnp.int32),
            pltpu.VMEM((_K,), jnp.int32),
            pltpu.VMEM((_K,), jnp.float32),
            pltpu.VMEM((_K,), jnp.float32),
            pltpu.VMEM((_K,), jnp.float32),
            pltpu.VMEM((_K,), jnp.float32),
            pltpu.VMEM((_K, CP), jnp.uint32),
            pltpu.VMEM((_K, CP), jnp.uint32),
            pltpu.VMEM((_K, CP), jnp.uint32),
            pltpu.VMEM((_K, CP), jnp.uint32),
            pltpu.VMEM((_K, CP), jnp.uint32),
            pltpu.SemaphoreType.DMA,
            pltpu.SemaphoreType.DMA,
            pltpu.SemaphoreType.DMA,
            pltpu.SemaphoreType.DMA,
            pltpu.SemaphoreType.DMA,
            pltpu.SemaphoreType.DMA,
        ]

    @functools.partial(
        pl.kernel,
        mesh=mesh,
        compiler_params=pltpu.CompilerParams(
            needs_layout_passes=False, use_tc_tiling_on_sc=False),
        out_type=jax.ShapeDtypeStruct((B, H, W, CP), jnp.uint32),
        scratch_types=slot_scratch() + slot_scratch(),
    )
    def warp(table_hbm, flow_hbm, out_hbm, *scr):
        slots = (scr[:20], scr[20:])
        wid = lax.axis_index("s") * _NC + lax.axis_index("c")
        base0 = wid * per_w
        lanes = lax.iota(jnp.int32, _L)

        def flow_cp(slot, ci):
            return pltpu.make_async_copy(
                flow_hbm.at[pl.ds(2 * (base0 + ci * _K), 2 * _K)],
                slot[0], slot[14])

        def out_cp(slot, ci):
            base = base0 + ci * _K
            row = base // W
            b = row // H
            return pltpu.make_async_copy(
                slot[13],
                out_hbm.at[b, row - b * H, pl.ds(base - row * W, _K), :],
                slot[19])

        def gather_cps(slot):
            return [pltpu.make_async_copy(table_hbm.at[slot[1 + k]],
                                          slot[9 + k], slot[15 + k])
                    for k in range(4)]

        def meta(slot, ci):
            flow_v = slot[0]
            base = base0 + ci * _K
            row = base // W
            b = row // H
            gy = row - b * H
            col0 = base - row * W
            gy_f = gy.astype(jnp.float32)
            bHW = b * (H * W)
            for g in range(_K // _L):
                rows2 = (g * _L) * 2 + lanes * 2
                fl_y = plsc.load_gather(flow_v, [rows2])
                fl_x = plsc.load_gather(flow_v, [rows2 + 1])
                qy = gy_f - fl_y
                gx = (col0 + g * _L) + lanes
                qx = gx.astype(jnp.float32) - fl_x
                qcy = jnp.minimum(jnp.maximum(qy, 0.0), float(H - 1))
                qcx = jnp.minimum(jnp.maximum(qx, 0.0), float(W - 1))
                fy = jnp.minimum(qcy.astype(jnp.int32), H - 2)
                fx = jnp.minimum(qcx.astype(jnp.int32), W - 2)
                ay = qcy - fy.astype(jnp.float32)
                ax = qcx - fx.astype(jnp.float32)
                i0 = (bHW + fx) + fy * W
                sl = pl.ds(g * _L, _L)
                slot[1][sl] = i0
                slot[2][sl] = i0 + 1
                slot[3][sl] = i0 + W
                slot[4][sl] = i0 + (W + 1)
                omy = 1.0 - ay
                omx = 1.0 - ax
                slot[5][sl] = omy * omx
                slot[6][sl] = omy * ax
                slot[7][sl] = ay * omx
                slot[8][sl] = ay * ax

        def blend(slot):
            tl_v, tr_v, bl_v, br_v, out_v = slot[9:14]

            def group(g, c):
                gsl = pl.ds(g * _L, _L)
                w00g = slot[5][gsl]
                w01g = slot[6][gsl]
                w10g = slot[7][gsl]
                w11g = slot[8][gsl]
                for p in range(_L):
                    i = g * _L + p

                    def splat_bf(wv, p=p):
                        w = jnp.full((_L,), wv[p], jnp.float32)
                        return plsc.pack(
                            w, w, format=plsc.PackFormat.INTERLEAVED)

                    w00 = splat_bf(w00g)
                    w01 = splat_bf(w01g)
                    w10 = splat_bf(w10g)
                    w11 = splat_bf(w11g)
                    for j in range(CP // _L):
                        s = pl.ds(j * _L, _L)
                        tl = plsc.bitcast(tl_v[i, s], jnp.bfloat16)
                        tr = plsc.bitcast(tr_v[i, s], jnp.bfloat16)
                        bl = plsc.bitcast(bl_v[i, s], jnp.bfloat16)
                        br = plsc.bitcast(br_v[i, s], jnp.bfloat16)
                        acc = (w00 * tl + w01 * tr) + (w10 * bl + w11 * br)
                        out_v[i, s] = plsc.bitcast(acc, jnp.uint32)
                return c

            lax.fori_loop(0, _K // _L, group, 0)

        def prep(slot, ci):
            flow_cp(slot, ci).wait()
            meta(slot, ci)
            for cp in gather_cps(slot):
                cp.start()

        A, Bt = slots
        flow_cp(A, 0).start()
        prep(A, 0)
        flow_cp(Bt, 1).start()

        def body(j, carry):
            c0 = 2 * j
            c1 = c0 + 1
            prep(Bt, c1)
            for cp in gather_cps(A):
                cp.wait()

            @pl.when(j > 0)
            def _():
                out_cp(A, c0 - 2).wait()

            blend(A)
            out_cp(A, c0).start()

            @pl.when(j < n_chunks // 2 - 1)
            def _():
                flow_cp(A, c0 + 2).start()
                prep(A, c0 + 2)

            for cp in gather_cps(Bt):
                cp.wait()

            @pl.when(j > 0)
            def _():
                out_cp(Bt, c1 - 2).wait()

            blend(Bt)
            out_cp(Bt, c1).start()

            @pl.when(j < n_chunks // 2 - 1)
            def _():
                flow_cp(Bt, c1 + 2).start()

            return carry

        lax.fori_loop(0, n_chunks // 2, body, 0)
        out_cp(A, n_chunks - 2).wait()
        out_cp(Bt, n_chunks - 1).wait()

    return warp


def kernel(frame_tail, flow):
    B, H, W, C = frame_tail.shape
    P = B * H * W
    t1 = lax.optimization_barrier(
        frame_tail.astype(jnp.bfloat16).reshape(-1))
    table = lax.bitcast_convert_type(
        t1.reshape(P, C // 2, 2), jnp.uint32)
    flow1 = flow.reshape(-1)
    out_u = _build_warp(B, H, W, C)(table, flow1)
    out_bf = lax.bitcast_convert_type(out_u, jnp.bfloat16)
    return out_bf.reshape(B, H, W, C).astype(jnp.float32)


# speedup vs baseline: 12.0121x; 12.0121x over previous
import functools

import jax
import jax.numpy as jnp
from jax import lax
from jax.experimental import pallas as pl
from jax.experimental.pallas import tpu as pltpu
from jax.experimental.pallas import tpu_sc as plsc

_NC = 2
_NS = 16
_NW = _NC * _NS
_L = 16
_K = 96


@functools.cache
def _build_warp(B, H, W, C):
    P = B * H * W
    CP = C // 2
    per_w = P // _NW
    n_chunks = per_w // _K
    assert per_w % _K == 0 and CP % _L == 0 and W % _K == 0
    assert n_chunks % 2 == 0

    mesh = plsc.VectorSubcoreMesh(core_axis_name="c", subcore_axis_name="s")

    def slot_scratch():
        return [
            pltpu.VMEM((2 * _K,), jnp.float32),
            pltpu.VMEM((_K,), jnp.int32),
            pltpu.VMEM((_K,), jnp.int32),
            pltpu.VMEM((_K,), jnp.int32),
            pltpu.VMEM((_K,), jnp.int32),
            pltpu.VMEM((_K,), jnp.float32),
            pltpu.VMEM((_K,), jnp.float32),
            pltpu.VMEM((_K,), jnp.float32),
            pltpu.VMEM((_K,), jnp.float32),
            pltpu.VMEM((_K, C), jnp.bfloat16),
            pltpu.VMEM((_K, C), jnp.bfloat16),
            pltpu.VMEM((_K, C), jnp.bfloat16),
            pltpu.VMEM((_K, C), jnp.bfloat16),
            pltpu.VMEM((_K, C), jnp.bfloat16),
            pltpu.SemaphoreType.DMA,
            pltpu.SemaphoreType.DMA,
            pltpu.SemaphoreType.DMA,
            pltpu.SemaphoreType.DMA,
            pltpu.SemaphoreType.DMA,
            pltpu.SemaphoreType.DMA,
        ]

    @functools.partial(
        pl.kernel,
        mesh=mesh,
        compiler_params=pltpu.CompilerParams(
            needs_layout_passes=False, use_tc_tiling_on_sc=False),
        out_type=jax.ShapeDtypeStruct((B, H, W, C), jnp.bfloat16),
        scratch_types=slot_scratch() + slot_scratch(),
    )
    def warp(table_hbm, flow_hbm, out_hbm, *scr):
        slots = (scr[:20], scr[20:])
        wid = lax.axis_index("s") * _NC + lax.axis_index("c")
        base0 = wid * per_w
        lanes = lax.iota(jnp.int32, _L)

        def flow_cp(slot, ci):
            return pltpu.make_async_copy(
                flow_hbm.at[pl.ds(2 * (base0 + ci * _K), 2 * _K)],
                slot[0], slot[14])

        def out_cp(slot, ci):
            base = base0 + ci * _K
            row = base // W
            b = row // H
            return pltpu.make_async_copy(
                slot[13],
                out_hbm.at[b, row - b * H, pl.ds(base - row * W, _K), :],
                slot[19])

        def gather_cps(slot):
            return [pltpu.make_async_copy(table_hbm.at[slot[1 + k]],
                                          slot[9 + k], slot[15 + k])
                    for k in range(4)]

        def meta(slot, ci):
            flow_v = slot[0]
            base = base0 + ci * _K
            row = base // W
            b = row // H
            gy = row - b * H
            col0 = base - row * W
            gy_f = gy.astype(jnp.float32)
            bHW = b * (H * W)
            for g in range(_K // _L):
                rows2 = (g * _L) * 2 + lanes * 2
                fl_y = plsc.load_gather(flow_v, [rows2])
                fl_x = plsc.load_gather(flow_v, [rows2 + 1])
                qy = gy_f - fl_y
                gx = (col0 + g * _L) + lanes
                qx = gx.astype(jnp.float32) - fl_x
                qcy = jnp.minimum(jnp.maximum(qy, 0.0), float(H - 1))
                qcx = jnp.minimum(jnp.maximum(qx, 0.0), float(W - 1))
                fy = jnp.minimum(qcy.astype(jnp.int32), H - 2)
                fx = jnp.minimum(qcx.astype(jnp.int32), W - 2)
                ay = qcy - fy.astype(jnp.float32)
                ax = qcx - fx.astype(jnp.float32)
                i0 = (bHW + fx) + fy * W
                sl = pl.ds(g * _L, _L)
                slot[1][sl] = i0
                slot[2][sl] = i0 + 1
                slot[3][sl] = i0 + W
                slot[4][sl] = i0 + (W + 1)
                omy = 1.0 - ay
                omx = 1.0 - ax
                slot[5][sl] = omy * omx
                slot[6][sl] = omy * ax
                slot[7][sl] = ay * omx
                slot[8][sl] = ay * ax

        def blend(slot):
            tl_v, tr_v, bl_v, br_v, out_v = slot[9:14]

            def group(g, c):
                gsl = pl.ds(g * _L, _L)
                w00g = slot[5][gsl]
                w01g = slot[6][gsl]
                w10g = slot[7][gsl]
                w11g = slot[8][gsl]
                for p in range(_L):
                    i = g * _L + p

                    def splat_bf(wv, p=p):
                        w = jnp.full((_L,), wv[p], jnp.float32)
                        return plsc.pack(
                            w, w, format=plsc.PackFormat.INTERLEAVED)

                    w00 = splat_bf(w00g)
                    w01 = splat_bf(w01g)
                    w10 = splat_bf(w10g)
                    w11 = splat_bf(w11g)
                    for j in range(C // (2 * _L)):
                        s = pl.ds(j * 2 * _L, 2 * _L)
                        tl = tl_v[i, s]
                        tr = tr_v[i, s]
                        bl = bl_v[i, s]
                        br = br_v[i, s]
                        out_v[i, s] = ((w00 * tl + w01 * tr)
                                       + (w10 * bl + w11 * br))
                return c

            lax.fori_loop(0, _K // _L, group, 0)

        def prep(slot, ci):
            flow_cp(slot, ci).wait()
            meta(slot, ci)
            for cp in gather_cps(slot):
                cp.start()

        A, Bt = slots
        flow_cp(A, 0).start()
        prep(A, 0)
        flow_cp(Bt, 1).start()

        def body(j, carry):
            c0 = 2 * j
            c1 = c0 + 1
            prep(Bt, c1)
            for cp in gather_cps(A):
                cp.wait()

            @pl.when(j > 0)
            def _():
                out_cp(A, c0 - 2).wait()

            blend(A)
            out_cp(A, c0).start()

            @pl.when(j < n_chunks // 2 - 1)
            def _():
                flow_cp(A, c0 + 2).start()
                prep(A, c0 + 2)

            for cp in gather_cps(Bt):
                cp.wait()

            @pl.when(j > 0)
            def _():
                out_cp(Bt, c1 - 2).wait()

            blend(Bt)
            out_cp(Bt, c1).start()

            @pl.when(j < n_chunks // 2 - 1)
            def _():
                flow_cp(Bt, c1 + 2).start()

            return carry

        lax.fori_loop(0, n_chunks // 2, body, 0)
        out_cp(A, n_chunks - 2).wait()
        out_cp(Bt, n_chunks - 1).wait()

    return warp


def kernel(frame_tail, flow):
    B, H, W, C = frame_tail.shape
    P = B * H * W
    t1 = lax.optimization_barrier(
        frame_tail.astype(jnp.bfloat16).reshape(-1))
    table = t1.reshape(P, C)
    flow1 = flow.reshape(-1)
    out_bf = _build_warp(B, H, W, C)(table, flow1)
    return out_bf.astype(jnp.float32)


# speedup vs baseline: 13.4252x; 1.1176x over previous
import functools

import jax
import jax.numpy as jnp
from jax import lax
from jax.experimental import pallas as pl
from jax.experimental.pallas import tpu as pltpu
from jax.experimental.pallas import tpu_sc as plsc

_NC = 2
_NS = 16
_NW = _NC * _NS
_L = 16
_K = 96


@functools.cache
def _build_warp(B, H, W, C):
    P = B * H * W
    CP = C // 2
    per_w = P // _NW
    n_chunks = per_w // _K
    assert per_w % _K == 0 and CP % _L == 0 and W % _K == 0
    assert n_chunks % 2 == 0

    mesh = plsc.VectorSubcoreMesh(core_axis_name="c", subcore_axis_name="s")

    def slot_scratch():
        return [
            pltpu.VMEM((2 * _K,), jnp.float32),
            pltpu.VMEM((_K,), jnp.int32),
            pltpu.VMEM((_K,), jnp.int32),
            pltpu.VMEM((_K,), jnp.int32),
            pltpu.VMEM((_K,), jnp.int32),
            pltpu.VMEM((_K,), jnp.float32),
            pltpu.VMEM((_K,), jnp.float32),
            pltpu.VMEM((_K,), jnp.float32),
            pltpu.VMEM((_K,), jnp.float32),
            pltpu.VMEM((_K, C), jnp.float32),
            pltpu.VMEM((_K, C), jnp.float32),
            pltpu.VMEM((_K, C), jnp.float32),
            pltpu.VMEM((_K, C), jnp.float32),
            pltpu.VMEM((_K, C), jnp.float32),
            pltpu.SemaphoreType.DMA,
            pltpu.SemaphoreType.DMA,
            pltpu.SemaphoreType.DMA,
            pltpu.SemaphoreType.DMA,
            pltpu.SemaphoreType.DMA,
            pltpu.SemaphoreType.DMA,
        ]

    @functools.partial(
        pl.kernel,
        mesh=mesh,
        compiler_params=pltpu.CompilerParams(
            needs_layout_passes=False, use_tc_tiling_on_sc=False),
        out_type=jax.ShapeDtypeStruct((B, H, W, C), jnp.float32),
        scratch_types=slot_scratch() + slot_scratch(),
    )
    def warp(table_hbm, flow_hbm, out_hbm, *scr):
        slots = (scr[:20], scr[20:])
        wid = lax.axis_index("s") * _NC + lax.axis_index("c")
        base0 = wid * per_w
        lanes = lax.iota(jnp.int32, _L)

        def flow_cp(slot, ci):
            return pltpu.make_async_copy(
                flow_hbm.at[pl.ds(2 * (base0 + ci * _K), 2 * _K)],
                slot[0], slot[14])

        def out_cp(slot, ci):
            base = base0 + ci * _K
            row = base // W
            b = row // H
            return pltpu.make_async_copy(
                slot[13],
                out_hbm.at[b, row - b * H, pl.ds(base - row * W, _K), :],
                slot[19])

        def gather_cps(slot):
            return [pltpu.make_async_copy(table_hbm.at[slot[1 + k]],
                                          slot[9 + k], slot[15 + k])
                    for k in range(4)]

        def meta(slot, ci):
            flow_v = slot[0]
            base = base0 + ci * _K
            row = base // W
            b = row // H
            gy = row - b * H
            col0 = base - row * W
            gy_f = gy.astype(jnp.float32)
            bHW = b * (H * W)
            for g in range(_K // _L):
                rows2 = (g * _L) * 2 + lanes * 2
                fl_y = plsc.load_gather(flow_v, [rows2])
                fl_x = plsc.load_gather(flow_v, [rows2 + 1])
                qy = gy_f - fl_y
                gx = (col0 + g * _L) + lanes
                qx = gx.astype(jnp.float32) - fl_x
                qcy = jnp.minimum(jnp.maximum(qy, 0.0), float(H - 1))
                qcx = jnp.minimum(jnp.maximum(qx, 0.0), float(W - 1))
                fy = jnp.minimum(qcy.astype(jnp.int32), H - 2)
                fx = jnp.minimum(qcx.astype(jnp.int32), W - 2)
                ay = qcy - fy.astype(jnp.float32)
                ax = qcx - fx.astype(jnp.float32)
                i0 = (bHW + fx) + fy * W
                sl = pl.ds(g * _L, _L)
                slot[1][sl] = i0
                slot[2][sl] = i0 + 1
                slot[3][sl] = i0 + W
                slot[4][sl] = i0 + (W + 1)
                omy = 1.0 - ay
                omx = 1.0 - ax
                slot[5][sl] = omy * omx
                slot[6][sl] = omy * ax
                slot[7][sl] = ay * omx
                slot[8][sl] = ay * ax

        def blend(slot):
            tl_v, tr_v, bl_v, br_v, out_v = slot[9:14]

            def group(g, c):
                gsl = pl.ds(g * _L, _L)
                w00g = slot[5][gsl]
                w01g = slot[6][gsl]
                w10g = slot[7][gsl]
                w11g = slot[8][gsl]
                for p in range(_L):
                    i = g * _L + p
                    w00 = jnp.full((_L,), w00g[p], jnp.float32)
                    w01 = jnp.full((_L,), w01g[p], jnp.float32)
                    w10 = jnp.full((_L,), w10g[p], jnp.float32)
                    w11 = jnp.full((_L,), w11g[p], jnp.float32)
                    for j in range(C // _L):
                        s = pl.ds(j * _L, _L)
                        out_v[i, s] = ((w00 * tl_v[i, s] + w01 * tr_v[i, s])
                                       + (w10 * bl_v[i, s] + w11 * br_v[i, s]))
                return c

            lax.fori_loop(0, _K // _L, group, 0)

        def prep(slot, ci):
            flow_cp(slot, ci).wait()
            meta(slot, ci)
            for cp in gather_cps(slot):
                cp.start()

        A, Bt = slots
        flow_cp(A, 0).start()
        prep(A, 0)
        flow_cp(Bt, 1).start()

        def body(j, carry):
            c0 = 2 * j
            c1 = c0 + 1
            prep(Bt, c1)
            for cp in gather_cps(A):
                cp.wait()

            @pl.when(j > 0)
            def _():
                out_cp(A, c0 - 2).wait()

            blend(A)
            out_cp(A, c0).start()

            @pl.when(j < n_chunks // 2 - 1)
            def _():
                flow_cp(A, c0 + 2).start()
                prep(A, c0 + 2)

            for cp in gather_cps(Bt):
                cp.wait()

            @pl.when(j > 0)
            def _():
                out_cp(Bt, c1 - 2).wait()

            blend(Bt)
            out_cp(Bt, c1).start()

            @pl.when(j < n_chunks // 2 - 1)
            def _():
                flow_cp(Bt, c1 + 2).start()

            return carry

        lax.fori_loop(0, n_chunks // 2, body, 0)
        out_cp(A, n_chunks - 2).wait()
        out_cp(Bt, n_chunks - 1).wait()

    return warp


def kernel(frame_tail, flow):
    B, H, W, C = frame_tail.shape
    P = B * H * W
    t1 = lax.optimization_barrier(frame_tail.reshape(-1))
    table = t1.reshape(P, C)
    flow1 = flow.reshape(-1)
    return _build_warp(B, H, W, C)(table, flow1)


# speedup vs baseline: 16.0423x; 1.1949x over previous
import functools

import jax
import jax.numpy as jnp
from jax import lax
from jax.experimental import pallas as pl
from jax.experimental.pallas import tpu as pltpu
from jax.experimental.pallas import tpu_sc as plsc

_NC = 2
_NS = 16
_NW = _NC * _NS
_L = 16
_K = 64
_CE = 128


@functools.cache
def _build_warp(B, H, W, C):
    P = B * H * W
    CP = C // 2
    per_w = P // _NW
    n_chunks = per_w // _K
    assert per_w % _K == 0 and CP % _L == 0 and W % _K == 0
    assert n_chunks % 2 == 0

    mesh = plsc.VectorSubcoreMesh(core_axis_name="c", subcore_axis_name="s")

    def slot_scratch():
        return [
            pltpu.VMEM((2 * _K,), jnp.float32),
            pltpu.VMEM((_K,), jnp.int32),
            pltpu.VMEM((_K,), jnp.int32),
            pltpu.VMEM((_K,), jnp.int32),
            pltpu.VMEM((_K,), jnp.int32),
            pltpu.VMEM((_K,), jnp.float32),
            pltpu.VMEM((_K,), jnp.float32),
            pltpu.VMEM((_K,), jnp.float32),
            pltpu.VMEM((_K,), jnp.float32),
            pltpu.VMEM((_K, _CE), jnp.float32),
            pltpu.VMEM((_K, _CE), jnp.float32),
            pltpu.VMEM((_K, _CE), jnp.float32),
            pltpu.VMEM((_K, _CE), jnp.float32),
            pltpu.VMEM((_K, _CE), jnp.float32),
            pltpu.SemaphoreType.DMA,
            pltpu.SemaphoreType.DMA,
            pltpu.SemaphoreType.DMA,
            pltpu.SemaphoreType.DMA,
            pltpu.SemaphoreType.DMA,
            pltpu.SemaphoreType.DMA,
        ]

    @functools.partial(
        pl.kernel,
        mesh=mesh,
        compiler_params=pltpu.CompilerParams(
            needs_layout_passes=False, use_tc_tiling_on_sc=False),
        out_type=jax.ShapeDtypeStruct((P, _CE), jnp.float32),
        scratch_types=slot_scratch() + slot_scratch(),
    )
    def warp(table_hbm, flow_hbm, out_hbm, *scr):
        slots = (scr[:20], scr[20:])
        wid = lax.axis_index("s") * _NC + lax.axis_index("c")
        base0 = wid * per_w
        lanes = lax.iota(jnp.int32, _L)

        def flow_cp(slot, ci):
            return pltpu.make_async_copy(
                flow_hbm.at[pl.ds(2 * (base0 + ci * _K), 2 * _K)],
                slot[0], slot[14])

        def out_cp(slot, ci):
            return pltpu.make_async_copy(
                slot[13], out_hbm.at[pl.ds(base0 + ci * _K, _K)], slot[19])

        def gather_cps(slot):
            return [pltpu.make_async_copy(table_hbm.at[slot[1 + k]],
                                          slot[9 + k], slot[15 + k])
                    for k in range(4)]

        def meta(slot, ci):
            flow_v = slot[0]
            base = base0 + ci * _K
            row = base // W
            b = row // H
            gy = row - b * H
            col0 = base - row * W
            gy_f = gy.astype(jnp.float32)
            bHW = b * (H * W)
            for g in range(_K // _L):
                rows2 = (g * _L) * 2 + lanes * 2
                fl_y = plsc.load_gather(flow_v, [rows2])
                fl_x = plsc.load_gather(flow_v, [rows2 + 1])
                qy = gy_f - fl_y
                gx = (col0 + g * _L) + lanes
                qx = gx.astype(jnp.float32) - fl_x
                qcy = jnp.minimum(jnp.maximum(qy, 0.0), float(H - 1))
                qcx = jnp.minimum(jnp.maximum(qx, 0.0), float(W - 1))
                fy = jnp.minimum(qcy.astype(jnp.int32), H - 2)
                fx = jnp.minimum(qcx.astype(jnp.int32), W - 2)
                ay = qcy - fy.astype(jnp.float32)
                ax = qcx - fx.astype(jnp.float32)
                i0 = (bHW + fx) + fy * W
                sl = pl.ds(g * _L, _L)
                slot[1][sl] = i0
                slot[2][sl] = i0 + 1
                slot[3][sl] = i0 + W
                slot[4][sl] = i0 + (W + 1)
                omy = 1.0 - ay
                omx = 1.0 - ax
                slot[5][sl] = omy * omx
                slot[6][sl] = omy * ax
                slot[7][sl] = ay * omx
                slot[8][sl] = ay * ax

        def blend(slot):
            tl_v, tr_v, bl_v, br_v, out_v = slot[9:14]

            def group(g, c):
                gsl = pl.ds(g * _L, _L)
                w00g = slot[5][gsl]
                w01g = slot[6][gsl]
                w10g = slot[7][gsl]
                w11g = slot[8][gsl]
                for p in range(_L):
                    i = g * _L + p
                    w00 = jnp.full((_L,), w00g[p], jnp.float32)
                    w01 = jnp.full((_L,), w01g[p], jnp.float32)
                    w10 = jnp.full((_L,), w10g[p], jnp.float32)
                    w11 = jnp.full((_L,), w11g[p], jnp.float32)
                    for j in range(C // _L):
                        s = pl.ds(j * _L, _L)
                        out_v[i, s] = ((w00 * tl_v[i, s] + w01 * tr_v[i, s])
                                       + (w10 * bl_v[i, s] + w11 * br_v[i, s]))
                return c

            lax.fori_loop(0, _K // _L, group, 0)

        def prep(slot, ci):
            flow_cp(slot, ci).wait()
            meta(slot, ci)
            for cp in gather_cps(slot):
                cp.start()

        A, Bt = slots
        flow_cp(A, 0).start()
        prep(A, 0)
        flow_cp(Bt, 1).start()

        def body(j, carry):
            c0 = 2 * j
            c1 = c0 + 1
            prep(Bt, c1)
            for cp in gather_cps(A):
                cp.wait()

            @pl.when(j > 0)
            def _():
                out_cp(A, c0 - 2).wait()

            blend(A)
            out_cp(A, c0).start()

            @pl.when(j < n_chunks // 2 - 1)
            def _():
                flow_cp(A, c0 + 2).start()
                prep(A, c0 + 2)

            for cp in gather_cps(Bt):
                cp.wait()

            @pl.when(j > 0)
            def _():
                out_cp(Bt, c1 - 2).wait()

            blend(Bt)
            out_cp(Bt, c1).start()

            @pl.when(j < n_chunks // 2 - 1)
            def _():
                flow_cp(Bt, c1 + 2).start()

            return carry

        lax.fori_loop(0, n_chunks // 2, body, 0)
        out_cp(A, n_chunks - 2).wait()
        out_cp(Bt, n_chunks - 1).wait()

    return warp


def kernel(frame_tail, flow):
    B, H, W, C = frame_tail.shape
    P = B * H * W
    CE = 128
    padded = jnp.pad(frame_tail, ((0, 0), (0, 0), (0, 0), (0, CE - C)))
    table = padded.reshape(P, CE)
    flow1 = flow.reshape(-1)
    out1 = _build_warp(B, H, W, C)(table, flow1)
    return out1.reshape(B, H, W, CE)[..., :C]
